# baseline (speedup 1.0000x reference)
import jax
import jax.numpy as jnp
from jax.experimental import pallas as pl


def _sdhgcn_body(adj_ref, x_ref, w_ref, out_ref):
    a = adj_ref[...].astype(jnp.float32)
    xw = jnp.dot(x_ref[...], w_ref[...],
                 preferred_element_type=jnp.float32)
    out_t = jax.lax.dot_general(
        xw, a, dimension_numbers=(((0,), (0,)), ((), ())),
        preferred_element_type=jnp.float32)
    deg = jnp.sum(adj_ref[...], axis=0).astype(jnp.float32)
    coeff = jax.lax.rsqrt(jnp.maximum(deg, 1.0))
    out_ref[...] = jnp.maximum(out_t * coeff[None, :], 0.0).T


def kernel(X, adj_matrix, weight):
    n, d_out = X.shape[0], weight.shape[1]
    return pl.pallas_call(
        _sdhgcn_body,
        out_shape=jax.ShapeDtypeStruct((n, d_out), jnp.float32),
    )(adj_matrix, X, weight)


# speedup vs baseline: 1.0060x; 1.0060x over previous
import jax
import jax.numpy as jnp
from jax.experimental import pallas as pl
from jax.experimental.pallas import tpu as pltpu

_BC = 512


def _sdhgcn_body(adj_ref, x_ref, w_ref, out_ref):
    a = adj_ref[...].astype(jnp.float32)
    xw = jnp.dot(x_ref[...], w_ref[...],
                 preferred_element_type=jnp.float32)
    out_t = jax.lax.dot_general(
        xw, a, dimension_numbers=(((0,), (0,)), ((), ())),
        preferred_element_type=jnp.float32)
    deg = jnp.sum(adj_ref[...], axis=0).astype(jnp.float32)
    coeff = jax.lax.rsqrt(jnp.maximum(deg, 1.0))
    out_ref[...] = jnp.maximum(out_t * coeff[None, :], 0.0).T


def kernel(X, adj_matrix, weight):
    n, d_in = X.shape
    d_out = weight.shape[1]
    nblk = n // _BC
    return pl.pallas_call(
        _sdhgcn_body,
        grid=(nblk,),
        in_specs=[
            pl.BlockSpec((n, _BC), lambda i: (0, i)),
            pl.BlockSpec((n, d_in), lambda i: (0, 0)),
            pl.BlockSpec((d_in, d_out), lambda i: (0, 0)),
        ],
        out_specs=pl.BlockSpec((_BC, d_out), lambda i: (i, 0)),
        out_shape=jax.ShapeDtypeStruct((n, d_out), jnp.float32),
        compiler_params=pltpu.CompilerParams(
            dimension_semantics=("arbitrary",)),
    )(adj_matrix, X, weight)
